# baseline (speedup 1.0000x reference)
import functools

import jax
import jax.numpy as jnp
from jax import lax
from jax.experimental import pallas as pl
from jax.experimental.pallas import tpu as pltpu
from jax.experimental.pallas import tpu_sc as plsc

EMB = 50
WINDOW = 5
BATCH = 16384
HIDDEN = 100
OUT = 64
VOCAB = 1000000

NUM_CORES = 2
NUM_SUBCORES = 16
NW = NUM_CORES * NUM_SUBCORES
NT = VOCAB // 8
K = 20
ROWS_PER_W = BATCH // NW
NCHW = ROWS_PER_W // 4


def _chunk_scalars(idx_v, j):
    g0 = idx_v[j, pl.ds(0, 16)]
    g1 = idx_v[j, pl.ds(4, 16)]
    return [g0[k] for k in range(16)] + [g1[k] for k in range(12, 16)]


def _enqueue_chunk(table_hbm, slab_v, scalars, sem, slot):
    for k in range(K):
        tidx = lax.shift_right_logical(scalars[k], 3)
        pltpu.async_copy(table_hbm.at[tidx], slab_v.at[slot, k], sem)


def _extract_chunk(slab_v, out_v, scalars, slot):
    for k in range(K):
        r = scalars[k] & 7
        w = k % 5
        row = slot * 4 + k // 5
        for c in (0, 16, 32, 34):
            out_v[w, row, pl.ds(c, 16)] = slab_v[slot, k, r, pl.ds(c, 16)]


def _sc_gather_body(idx_hbm, table_hbm, out_hbm, idx_v, slab_v, out_v,
                    sem0, sem1):
    wid = lax.axis_index("s") * NUM_CORES + lax.axis_index("c")
    wbase = wid * ROWS_PER_W
    pltpu.sync_copy(
        idx_hbm.at[pl.ds(pl.multiple_of(wid * NCHW, NCHW), NCHW)], idx_v)

    def _wait(slot, sem):
        pltpu.make_async_copy(
            table_hbm.at[pl.ds(0, K)], slab_v.at[slot], sem).wait()

    sc0 = _chunk_scalars(idx_v, 0)
    _enqueue_chunk(table_hbm, slab_v, sc0, sem0, 0)

    def step(g, carry):
        sc1 = _chunk_scalars(idx_v, 2 * g + 1)
        _enqueue_chunk(table_hbm, slab_v, sc1, sem1, 1)
        sc0 = _chunk_scalars(idx_v, 2 * g)
        _wait(0, sem0)
        _extract_chunk(slab_v, out_v, sc0, 0)

        @pl.when(g < NCHW // 2 - 1)
        def _():
            sc2 = _chunk_scalars(idx_v, 2 * g + 2)
            _enqueue_chunk(table_hbm, slab_v, sc2, sem0, 0)

        _wait(1, sem1)
        _extract_chunk(slab_v, out_v, sc1, 1)
        base = pl.multiple_of(wbase + g * 8, 8)
        for w in range(WINDOW):
            pltpu.sync_copy(out_v.at[w], out_hbm.at[w, pl.ds(base, 8)])
        return carry

    lax.fori_loop(0, NCHW // 2, step, 0)


@functools.cache
def _build_sc_gather():
    mesh = plsc.VectorSubcoreMesh(core_axis_name="c", subcore_axis_name="s")
    return pl.kernel(
        _sc_gather_body,
        out_type=jax.ShapeDtypeStruct((WINDOW, BATCH, EMB), jnp.float32),
        mesh=mesh,
        scratch_types=[
            pltpu.VMEM((NCHW, K), jnp.int32),
            pltpu.VMEM((2, K, 8, EMB), jnp.float32),
            pltpu.VMEM((WINDOW, 8, EMB), jnp.float32),
            pltpu.SemaphoreType.DMA,
            pltpu.SemaphoreType.DMA,
        ],
    )


BLOCK_B = 2048


def _mlp_body(e0, e1, e2, e3, e4, wh_ref, bh_ref, wo_ref, bo_ref, out_ref):
    acc = bh_ref[...]
    for w, e_ref in enumerate((e0, e1, e2, e3, e4)):
        acc = acc + jnp.dot(
            e_ref[...], wh_ref[pl.ds(w * EMB, EMB), :],
            preferred_element_type=jnp.float32)
    h = jnp.tanh(acc)
    logits = (
        jnp.dot(h, wo_ref[...], preferred_element_type=jnp.float32)
        + bo_ref[...])
    m = jnp.max(logits, axis=1, keepdims=True)
    ex = jnp.exp(logits - m)
    out_ref[...] = ex / jnp.sum(ex, axis=1, keepdims=True)


_e_spec = pl.BlockSpec((BLOCK_B, EMB), lambda i: (i, 0))
_mlp = pl.pallas_call(
    _mlp_body,
    grid=(BATCH // BLOCK_B,),
    in_specs=[
        _e_spec, _e_spec, _e_spec, _e_spec, _e_spec,
        pl.BlockSpec((WINDOW * EMB, HIDDEN), lambda i: (0, 0)),
        pl.BlockSpec((1, HIDDEN), lambda i: (0, 0)),
        pl.BlockSpec((HIDDEN, OUT), lambda i: (0, 0)),
        pl.BlockSpec((1, OUT), lambda i: (0, 0)),
    ],
    out_specs=pl.BlockSpec((BLOCK_B, OUT), lambda i: (i, 0)),
    out_shape=jax.ShapeDtypeStruct((BATCH, OUT), jnp.float32),
)


def kernel(x, emb_table, W_h, b_h, W_o, b_o):
    t3 = emb_table.reshape(NT, 8, EMB)
    idx = x.reshape(NW * NCHW, K)
    e5 = _build_sc_gather()(idx, t3)
    return _mlp(e5[0], e5[1], e5[2], e5[3], e5[4],
                W_h, b_h.reshape(1, HIDDEN), W_o, b_o.reshape(1, OUT))
